# baseline (speedup 1.0000x reference)
import functools

import jax
import jax.numpy as jnp
from jax import lax
from jax.experimental import pallas as pl
from jax.experimental.pallas import tpu as pltpu
from jax.experimental.pallas import tpu_sc as plsc

N = 50000
E = 800000
NUM_NUMERIC = 16
NUM_CAT = 8
CAT_SIZE = 10000
EMBED = 32
HIDDEN = 64

NC = 2
NS = 16
C = 128

NP = 50176
BLK = 512
NBLK = NP // BLK
GP = NP * NUM_CAT
G_PER_W = GP // (NC * NS)
EP = 802816
E_PER_TILE = EP // NS
E_PER_W = EP // (NC * NS)
ROWS_PER_TILE = NP // NS


_sc_mesh = plsc.VectorSubcoreMesh(core_axis_name="c", subcore_axis_name="s")
_sc_params = pltpu.CompilerParams(use_tc_tiling_on_sc=False)


NBE = G_PER_W // C


@functools.partial(
    pl.kernel,
    out_type=jax.ShapeDtypeStruct((GP, EMBED), jnp.float32),
    mesh=_sc_mesh,
    scratch_types=[
        pltpu.VMEM((C,), jnp.int32),
        pltpu.VMEM((C,), jnp.int32),
        pltpu.VMEM((C, EMBED), jnp.float32),
        pltpu.VMEM((C, EMBED), jnp.float32),
        pltpu.SemaphoreType.DMA,
        pltpu.SemaphoreType.DMA,
        pltpu.SemaphoreType.DMA,
        pltpu.SemaphoreType.DMA,
        pltpu.SemaphoreType.DMA,
        pltpu.SemaphoreType.DMA,
    ],
    compiler_params=_sc_params,
)
def _emb_gather(tab_hbm, ids_hbm, out_hbm, eidx0, eidx1, erows0, erows1,
                isem0, isem1, gsem0, gsem1, osem0, osem1):
    c = lax.axis_index("c")
    s = lax.axis_index("s")
    base = (s * NC + c) * G_PER_W
    eidx = (eidx0, eidx1)
    erows = (erows0, erows1)
    isem = (isem0, isem1)
    gsem = (gsem0, gsem1)
    osem = (osem0, osem1)

    def idx_fire(b, p):
        off = base + b * C
        pltpu.async_copy(ids_hbm.at[pl.ds(off, C)], eidx[p], isem[p])

    def gat_fire(b, p):
        pltpu.make_async_copy(ids_hbm.at[pl.ds(0, C)], eidx[p], isem[p]).wait()

        if not (isinstance(b, int) and b < 2):
            @pl.when(b >= 2)
            def _():
                pltpu.make_async_copy(
                    erows[p], out_hbm.at[pl.ds(0, C)], osem[p]).wait()

        pltpu.async_copy(tab_hbm.at[eidx[p]], erows[p], gsem[p])

    def drain(b, p):
        off = base + b * C
        pltpu.make_async_copy(tab_hbm.at[pl.ds(0, C)], erows[p], gsem[p]).wait()
        pltpu.async_copy(erows[p], out_hbm.at[pl.ds(off, C)], osem[p])

    idx_fire(0, 0)
    idx_fire(1, 1)
    gat_fire(0, 0)

    def body(g, carry):
        b0 = 2 * g

        drain(b0, 0)

        @pl.when(b0 + 2 < NBE)
        def _():
            idx_fire(b0 + 2, 0)

        @pl.when(b0 + 1 < NBE)
        def _():
            gat_fire(b0 + 1, 1)

        @pl.when(b0 + 1 < NBE)
        def _():
            drain(b0 + 1, 1)

        @pl.when(b0 + 3 < NBE)
        def _():
            idx_fire(b0 + 3, 1)

        @pl.when(b0 + 2 < NBE)
        def _():
            gat_fire(b0 + 2, 0)

        return carry

    lax.fori_loop(0, (NBE + 1) // 2, body, 0)
    pltpu.make_async_copy(erows0, out_hbm.at[pl.ds(0, C)], osem0).wait()
    pltpu.make_async_copy(erows1, out_hbm.at[pl.ds(0, C)], osem1).wait()


DEGW = 32

NBD = E_PER_W // C


@functools.partial(
    pl.kernel,
    out_type=jax.ShapeDtypeStruct((NC, NP, DEGW), jnp.float32),
    mesh=_sc_mesh,
    scratch_types=[
        pltpu.VMEM_SHARED((NP, DEGW), jnp.float32),
        pltpu.VMEM((C,), jnp.int32),
        pltpu.VMEM((C,), jnp.int32),
        pltpu.VMEM((C, DEGW), jnp.float32),
        pltpu.SemaphoreType.DMA,
        pltpu.SemaphoreType.DMA,
    ],
    compiler_params=_sc_params,
)
def _deg_count(dst_hbm, ones_hbm, zeros_hbm, out_hbm, deg_sp,
               didx0, didx1, ones_v, sem0, sem1):
    c = lax.axis_index("c")
    s = lax.axis_index("s")
    pltpu.sync_copy(ones_hbm, ones_v)
    pltpu.sync_copy(zeros_hbm, deg_sp.at[pl.ds(s * ROWS_PER_TILE, ROWS_PER_TILE), :])
    plsc.subcore_barrier()
    base = (s * NC + c) * E_PER_W
    didx = (didx0, didx1)
    sems = (sem0, sem1)

    def fire(k, p):
        off = base + k * C
        pltpu.async_copy(dst_hbm.at[pl.ds(off, C)], didx[p], sems[p])

    def drain(p):
        pltpu.make_async_copy(dst_hbm.at[pl.ds(0, C)], didx[p], sems[p]).wait()
        pltpu.sync_copy(ones_v, deg_sp.at[didx[p]], add=True)

    fire(0, 0)

    def body(g, carry):
        k0 = 2 * g

        @pl.when(k0 + 1 < NBD)
        def _():
            fire(k0 + 1, 1)

        drain(0)

        @pl.when(k0 + 2 < NBD)
        def _():
            fire(k0 + 2, 0)

        @pl.when(k0 + 1 < NBD)
        def _():
            drain(1)

        return carry

    lax.fori_loop(0, (NBD + 1) // 2, body, 0)
    plsc.subcore_barrier()
    pltpu.sync_copy(
        deg_sp.at[pl.ds(s * ROWS_PER_TILE, ROWS_PER_TILE), :],
        out_hbm.at[c, pl.ds(s * ROWS_PER_TILE, ROWS_PER_TILE), :],
    )


KI = 2
EBLK = KI * C
NBK = E_PER_TILE // EBLK


@functools.partial(
    pl.kernel,
    out_type=jax.ShapeDtypeStruct((NC, NP, EMBED), jnp.float32),
    mesh=_sc_mesh,
    scratch_types=[
        pltpu.VMEM_SHARED((NP, EMBED), jnp.float32),
        pltpu.VMEM((EBLK,), jnp.int32),
        pltpu.VMEM((EBLK,), jnp.int32),
        pltpu.VMEM((C,), jnp.int32),
        pltpu.VMEM((C,), jnp.int32),
        pltpu.VMEM((C,), jnp.int32),
        pltpu.VMEM((C,), jnp.int32),
        pltpu.VMEM((KI, C, EMBED), jnp.float32),
        pltpu.VMEM((KI, C, EMBED), jnp.float32),
        pltpu.SemaphoreType.DMA,
        pltpu.SemaphoreType.DMA,
        pltpu.SemaphoreType.DMA,
        pltpu.SemaphoreType.DMA,
        pltpu.SemaphoreType.DMA,
        pltpu.SemaphoreType.DMA,
    ],
    compiler_params=_sc_params,
)
def _edge_agg(ht_hbm, src2_hbm, dst_hbm, zeros_hbm, out_hbm,
              agg_sp, sidx0, sidx1, d00, d01, d10, d11, rows0, rows1,
              ssem0, ssem1, dsem0, dsem1, gsem0, gsem1):
    c = lax.axis_index("c")
    s = lax.axis_index("s")
    pltpu.sync_copy(zeros_hbm, agg_sp.at[pl.ds(s * ROWS_PER_TILE, ROWS_PER_TILE), :])
    plsc.subcore_barrier()
    base = s * E_PER_TILE
    sidx = (sidx0, sidx1)
    didx = ((d00, d01), (d10, d11))
    rows = (rows0, rows1)
    ssem = (ssem0, ssem1)
    dsem = (dsem0, dsem1)
    gsem = (gsem0, gsem1)

    def idx_fire(b, p):
        off = base + b * EBLK
        pltpu.async_copy(src2_hbm.at[c, pl.ds(off, EBLK)], sidx[p], ssem[p])
        for j in range(KI):
            pltpu.async_copy(
                dst_hbm.at[pl.ds(off + j * C, C)], didx[p][j], dsem[p])

    def gat_fire(b, p):
        pltpu.make_async_copy(
            src2_hbm.at[c, pl.ds(0, EBLK)], sidx[p], ssem[p]).wait()
        for j in range(KI):
            pltpu.async_copy(
                ht_hbm.at[sidx[p].at[pl.ds(j * C, C)]], rows[p].at[j], gsem[p])

    def drain_scatter(b, p):
        for j in range(KI):
            pltpu.make_async_copy(
                ht_hbm.at[pl.ds(0, C)], rows[p].at[j], gsem[p]).wait()
        for j in range(KI):
            pltpu.make_async_copy(
                dst_hbm.at[pl.ds(0, C)], didx[p][j], dsem[p]).wait()
        for j in range(KI):
            pltpu.sync_copy(rows[p].at[j], agg_sp.at[didx[p][j]], add=True)

    idx_fire(0, 0)
    idx_fire(1, 1)
    gat_fire(0, 0)

    def body(g, carry):
        b0 = 2 * g

        drain_scatter(b0, 0)

        @pl.when(b0 + 2 < NBK)
        def _():
            idx_fire(b0 + 2, 0)

        @pl.when(b0 + 1 < NBK)
        def _():
            gat_fire(b0 + 1, 1)

        @pl.when(b0 + 1 < NBK)
        def _():
            drain_scatter(b0 + 1, 1)

        @pl.when(b0 + 3 < NBK)
        def _():
            idx_fire(b0 + 3, 1)

        @pl.when(b0 + 2 < NBK)
        def _():
            gat_fire(b0 + 2, 0)

        return carry

    lax.fori_loop(0, (NBK + 1) // 2, body, 0)
    plsc.subcore_barrier()
    pltpu.sync_copy(
        agg_sp.at[pl.ds(s * ROWS_PER_TILE, ROWS_PER_TILE), :],
        out_hbm.at[c, pl.ds(s * ROWS_PER_TILE, ROWS_PER_TILE), :],
    )


def _enc_body(x_ref, cat_ref, wn_ref, bn_ref, win_n_ref, win_c_ref, bin_ref,
              out_ref):
    num = jnp.maximum(
        jnp.dot(x_ref[...], wn_ref[...], preferred_element_type=jnp.float32)
        + bn_ref[...], 0.0)
    h = (jnp.dot(num, win_n_ref[...], preferred_element_type=jnp.float32)
         + jnp.dot(cat_ref[...], win_c_ref[...],
                   preferred_element_type=jnp.float32)
         + bin_ref[...])
    h = jnp.maximum(h, 0.0)
    out_ref[0] = h[:, :EMBED]
    out_ref[1] = h[:, EMBED:]


_encoder = pl.pallas_call(
    _enc_body,
    grid=(NBLK,),
    in_specs=[
        pl.BlockSpec((BLK, NUM_NUMERIC), lambda i: (i, 0)),
        pl.BlockSpec((BLK, NUM_CAT * EMBED), lambda i: (i, 0)),
        pl.BlockSpec((NUM_NUMERIC, EMBED), lambda i: (0, 0)),
        pl.BlockSpec((1, EMBED), lambda i: (0, 0)),
        pl.BlockSpec((EMBED, HIDDEN), lambda i: (0, 0)),
        pl.BlockSpec((NUM_CAT * EMBED, HIDDEN), lambda i: (0, 0)),
        pl.BlockSpec((1, HIDDEN), lambda i: (0, 0)),
    ],
    out_specs=pl.BlockSpec((2, BLK, EMBED), lambda i: (0, i, 0)),
    out_shape=jax.ShapeDtypeStruct((2, NP, EMBED), jnp.float32),
)


def _layer_body(with_head, agg_ref, deg_ref, hres_ref, wg_ref, bg_ref,
                gam_ref, bet_ref, *rest):
    if with_head:
        wo_ref, bo_ref, out_ref, outc_ref = rest
    else:
        (out_ref,) = rest
    agg = jnp.concatenate([agg_ref[0], agg_ref[1]], axis=1)
    deg = deg_ref[0][:, :1] + deg_ref[1][:, :1]
    hmp = agg * (1.0 / jnp.maximum(deg, 1.0))
    hr = jnp.concatenate([hres_ref[0], hres_ref[1]], axis=1)
    t = jnp.maximum(
        jnp.dot(hmp, wg_ref[...], preferred_element_type=jnp.float32)
        + bg_ref[...], 0.0) + hr
    mu = jnp.mean(t, axis=1, keepdims=True)
    var = jnp.mean((t - mu) ** 2, axis=1, keepdims=True)
    h = (t - mu) * lax.rsqrt(var + 1e-5) * gam_ref[...] + bet_ref[...]
    out_ref[0] = h[:, :EMBED]
    out_ref[1] = h[:, EMBED:]
    if with_head:
        outc_ref[...] = (
            jnp.dot(h, wo_ref[...], preferred_element_type=jnp.float32)
            + bo_ref[...])


def _make_layer(with_head):
    in_specs = [
        pl.BlockSpec((2, BLK, EMBED), lambda i: (0, i, 0)),
        pl.BlockSpec((2, BLK, DEGW), lambda i: (0, i, 0)),
        pl.BlockSpec((2, BLK, EMBED), lambda i: (0, i, 0)),
        pl.BlockSpec((HIDDEN, HIDDEN), lambda i: (0, 0)),
        pl.BlockSpec((1, HIDDEN), lambda i: (0, 0)),
        pl.BlockSpec((1, HIDDEN), lambda i: (0, 0)),
        pl.BlockSpec((1, HIDDEN), lambda i: (0, 0)),
    ]
    out_specs = pl.BlockSpec((2, BLK, EMBED), lambda i: (0, i, 0))
    out_shape = jax.ShapeDtypeStruct((2, NP, EMBED), jnp.float32)
    if with_head:
        in_specs += [
            pl.BlockSpec((HIDDEN, 1), lambda i: (0, 0)),
            pl.BlockSpec((1, 1), lambda i: (0, 0)),
        ]
        out_specs = [out_specs, pl.BlockSpec((BLK, 1), lambda i: (i, 0))]
        out_shape = [out_shape, jax.ShapeDtypeStruct((NP, 1), jnp.float32)]
    return pl.pallas_call(
        functools.partial(_layer_body, with_head),
        grid=(NBLK,),
        in_specs=in_specs,
        out_specs=out_specs,
        out_shape=out_shape,
    )


_layer1 = _make_layer(False)
_layer2 = _make_layer(True)


def kernel(x_num, x_cat, edge_index, W_num, b_num, emb_tables, W_in, b_in,
           W_g1, b_g1, W_g2, b_g2, gamma1, beta1, gamma2, beta2, W_out, b_out):
    f32 = jnp.float32
    ids = jnp.clip(x_cat, 0, CAT_SIZE) + (
        jnp.arange(NUM_CAT, dtype=jnp.int32) * (CAT_SIZE + 1))
    ids = jnp.pad(ids, ((0, NP - N), (0, 0))).reshape(-1)
    flat_tab = emb_tables.reshape(NUM_CAT * (CAT_SIZE + 1), EMBED)
    x_num_p = jnp.pad(x_num, ((0, NP - N), (0, 0)))

    src = jnp.pad(edge_index[0], (0, EP - E))
    dst = jnp.pad(edge_index[1], (0, EP - E), constant_values=N)
    src2 = jnp.stack([src, src + NP])

    zeros_rows = jnp.zeros((ROWS_PER_TILE, EMBED), f32)
    zeros_col = jnp.zeros((ROWS_PER_TILE, DEGW), f32)
    ones_col = jnp.ones((C, DEGW), f32)

    cat_embed = _emb_gather(flat_tab, ids).reshape(NP, NUM_CAT * EMBED)
    h0 = _encoder(x_num_p, cat_embed, W_num, b_num.reshape(1, EMBED),
                  W_in[:EMBED], W_in[EMBED:], b_in.reshape(1, HIDDEN))
    deg2 = _deg_count(dst, ones_col, zeros_col)

    agg1 = _edge_agg(h0.reshape(2 * NP, EMBED), src2, dst, zeros_rows)
    h1 = _layer1(agg1, deg2, h0, W_g1, b_g1.reshape(1, HIDDEN),
                 gamma1.reshape(1, HIDDEN), beta1.reshape(1, HIDDEN))

    agg2 = _edge_agg(h1.reshape(2 * NP, EMBED), src2, dst, zeros_rows)
    _, outc = _layer2(agg2, deg2, h1, W_g2, b_g2.reshape(1, HIDDEN),
                      gamma2.reshape(1, HIDDEN), beta2.reshape(1, HIDDEN),
                      W_out, b_out.reshape(1, 1))
    return outc[:N, 0]


# speedup vs baseline: 1.0475x; 1.0475x over previous
import functools

import jax
import jax.numpy as jnp
from jax import lax
from jax.experimental import pallas as pl
from jax.experimental.pallas import tpu as pltpu
from jax.experimental.pallas import tpu_sc as plsc

N = 50000
E = 800000
NUM_NUMERIC = 16
NUM_CAT = 8
CAT_SIZE = 10000
EMBED = 32
HIDDEN = 64

NC = 2
NS = 16
C = 128

NP = 50176
BLK = 512
NBLK = NP // BLK
GP = NP * NUM_CAT
G_PER_W = GP // (NC * NS)
EP = 802816
E_PER_TILE = EP // NS
E_PER_W = EP // (NC * NS)
ROWS_PER_TILE = NP // NS


_sc_mesh = plsc.VectorSubcoreMesh(core_axis_name="c", subcore_axis_name="s")
_sc_params = pltpu.CompilerParams(use_tc_tiling_on_sc=False)


NBE = G_PER_W // C


@functools.partial(
    pl.kernel,
    out_type=jax.ShapeDtypeStruct((GP, EMBED), jnp.float32),
    mesh=_sc_mesh,
    scratch_types=[
        pltpu.VMEM((C,), jnp.int32),
        pltpu.VMEM((C,), jnp.int32),
        pltpu.VMEM((C, EMBED), jnp.float32),
        pltpu.VMEM((C, EMBED), jnp.float32),
        pltpu.SemaphoreType.DMA,
        pltpu.SemaphoreType.DMA,
        pltpu.SemaphoreType.DMA,
        pltpu.SemaphoreType.DMA,
        pltpu.SemaphoreType.DMA,
        pltpu.SemaphoreType.DMA,
    ],
    compiler_params=_sc_params,
)
def _emb_gather(tab_hbm, ids_hbm, out_hbm, eidx0, eidx1, erows0, erows1,
                isem0, isem1, gsem0, gsem1, osem0, osem1):
    c = lax.axis_index("c")
    s = lax.axis_index("s")
    base = (s * NC + c) * G_PER_W
    eidx = (eidx0, eidx1)
    erows = (erows0, erows1)
    isem = (isem0, isem1)
    gsem = (gsem0, gsem1)
    osem = (osem0, osem1)

    def idx_fire(b, p):
        off = base + b * C
        pltpu.async_copy(ids_hbm.at[pl.ds(off, C)], eidx[p], isem[p])

    def gat_fire(b, p):
        pltpu.make_async_copy(ids_hbm.at[pl.ds(0, C)], eidx[p], isem[p]).wait()

        if not (isinstance(b, int) and b < 2):
            @pl.when(b >= 2)
            def _():
                pltpu.make_async_copy(
                    erows[p], out_hbm.at[pl.ds(0, C)], osem[p]).wait()

        pltpu.async_copy(tab_hbm.at[eidx[p]], erows[p], gsem[p])

    def drain(b, p):
        off = base + b * C
        pltpu.make_async_copy(tab_hbm.at[pl.ds(0, C)], erows[p], gsem[p]).wait()
        pltpu.async_copy(erows[p], out_hbm.at[pl.ds(off, C)], osem[p])

    idx_fire(0, 0)
    idx_fire(1, 1)
    gat_fire(0, 0)

    def body(g, carry):
        b0 = 2 * g

        drain(b0, 0)

        @pl.when(b0 + 2 < NBE)
        def _():
            idx_fire(b0 + 2, 0)

        @pl.when(b0 + 1 < NBE)
        def _():
            gat_fire(b0 + 1, 1)

        @pl.when(b0 + 1 < NBE)
        def _():
            drain(b0 + 1, 1)

        @pl.when(b0 + 3 < NBE)
        def _():
            idx_fire(b0 + 3, 1)

        @pl.when(b0 + 2 < NBE)
        def _():
            gat_fire(b0 + 2, 0)

        return carry

    lax.fori_loop(0, (NBE + 1) // 2, body, 0)
    pltpu.make_async_copy(erows0, out_hbm.at[pl.ds(0, C)], osem0).wait()
    pltpu.make_async_copy(erows1, out_hbm.at[pl.ds(0, C)], osem1).wait()


DEGW = 32

NBD = E_PER_W // C


@functools.partial(
    pl.kernel,
    out_type=jax.ShapeDtypeStruct((NC, NP, DEGW), jnp.float32),
    mesh=_sc_mesh,
    scratch_types=[
        pltpu.VMEM_SHARED((NP, DEGW), jnp.float32),
        pltpu.VMEM((C,), jnp.int32),
        pltpu.VMEM((C,), jnp.int32),
        pltpu.VMEM((C, DEGW), jnp.float32),
        pltpu.SemaphoreType.DMA,
        pltpu.SemaphoreType.DMA,
    ],
    compiler_params=_sc_params,
)
def _deg_count(dst_hbm, ones_hbm, zeros_hbm, out_hbm, deg_sp,
               didx0, didx1, ones_v, sem0, sem1):
    c = lax.axis_index("c")
    s = lax.axis_index("s")
    pltpu.sync_copy(ones_hbm, ones_v)
    pltpu.sync_copy(zeros_hbm, deg_sp.at[pl.ds(s * ROWS_PER_TILE, ROWS_PER_TILE), :])
    plsc.subcore_barrier()
    base = (s * NC + c) * E_PER_W
    didx = (didx0, didx1)
    sems = (sem0, sem1)

    def fire(k, p):
        off = base + k * C
        pltpu.async_copy(dst_hbm.at[pl.ds(off, C)], didx[p], sems[p])

    def drain(p):
        pltpu.make_async_copy(dst_hbm.at[pl.ds(0, C)], didx[p], sems[p]).wait()
        pltpu.sync_copy(ones_v, deg_sp.at[didx[p]], add=True)

    fire(0, 0)

    def body(g, carry):
        k0 = 2 * g

        @pl.when(k0 + 1 < NBD)
        def _():
            fire(k0 + 1, 1)

        drain(0)

        @pl.when(k0 + 2 < NBD)
        def _():
            fire(k0 + 2, 0)

        @pl.when(k0 + 1 < NBD)
        def _():
            drain(1)

        return carry

    lax.fori_loop(0, (NBD + 1) // 2, body, 0)
    plsc.subcore_barrier()
    pltpu.sync_copy(
        deg_sp.at[pl.ds(s * ROWS_PER_TILE, ROWS_PER_TILE), :],
        out_hbm.at[c, pl.ds(s * ROWS_PER_TILE, ROWS_PER_TILE), :],
    )


KI = 2
EBLK = KI * C
NBK = E_PER_TILE // EBLK


@functools.partial(
    pl.kernel,
    out_type=jax.ShapeDtypeStruct((NC, NP, EMBED), jnp.float32),
    mesh=_sc_mesh,
    scratch_types=[
        pltpu.VMEM_SHARED((NP, EMBED), jnp.float32),
        pltpu.VMEM((EBLK,), jnp.int32),
        pltpu.VMEM((EBLK,), jnp.int32),
        pltpu.VMEM((C,), jnp.int32),
        pltpu.VMEM((C,), jnp.int32),
        pltpu.VMEM((C,), jnp.int32),
        pltpu.VMEM((C,), jnp.int32),
        pltpu.VMEM((KI, C, EMBED), jnp.float32),
        pltpu.VMEM((KI, C, EMBED), jnp.float32),
        pltpu.SemaphoreType.DMA,
        pltpu.SemaphoreType.DMA,
        pltpu.SemaphoreType.DMA,
        pltpu.SemaphoreType.DMA,
        pltpu.SemaphoreType.DMA,
        pltpu.SemaphoreType.DMA,
    ],
    compiler_params=_sc_params,
)
def _edge_agg(ht_hbm, src2_hbm, dst_hbm, zeros_hbm, out_hbm,
              agg_sp, sidx0, sidx1, d00, d01, d10, d11, rows0, rows1,
              ssem0, ssem1, dsem0, dsem1, gsem0, gsem1):
    c = lax.axis_index("c")
    s = lax.axis_index("s")
    pltpu.sync_copy(zeros_hbm, agg_sp.at[pl.ds(s * ROWS_PER_TILE, ROWS_PER_TILE), :])
    plsc.subcore_barrier()
    base = s * E_PER_TILE
    sidx = (sidx0, sidx1)
    didx = ((d00, d01), (d10, d11))
    rows = (rows0, rows1)
    ssem = (ssem0, ssem1)
    dsem = (dsem0, dsem1)
    gsem = (gsem0, gsem1)

    def idx_fire(b, p):
        off = base + b * EBLK
        pltpu.async_copy(src2_hbm.at[c, pl.ds(off, EBLK)], sidx[p], ssem[p])
        for j in range(KI):
            pltpu.async_copy(
                dst_hbm.at[pl.ds(off + j * C, C)], didx[p][j], dsem[p])

    def gat_fire(b, p):
        pltpu.make_async_copy(
            src2_hbm.at[c, pl.ds(0, EBLK)], sidx[p], ssem[p]).wait()
        for j in range(KI):
            pltpu.async_copy(
                ht_hbm.at[sidx[p].at[pl.ds(j * C, C)]], rows[p].at[j], gsem[p])

    def drain_scatter(b, p):
        for j in range(KI):
            pltpu.make_async_copy(
                ht_hbm.at[pl.ds(0, C)], rows[p].at[j], gsem[p]).wait()
        for j in range(KI):
            pltpu.make_async_copy(
                dst_hbm.at[pl.ds(0, C)], didx[p][j], dsem[p]).wait()
        for j in range(KI):
            pltpu.sync_copy(rows[p].at[j], agg_sp.at[didx[p][j]], add=True)

    idx_fire(0, 0)
    idx_fire(1, 1)
    gat_fire(0, 0)

    def body(g, carry):
        b0 = 2 * g

        drain_scatter(b0, 0)

        @pl.when(b0 + 2 < NBK)
        def _():
            idx_fire(b0 + 2, 0)

        @pl.when(b0 + 1 < NBK)
        def _():
            gat_fire(b0 + 1, 1)

        @pl.when(b0 + 1 < NBK)
        def _():
            drain_scatter(b0 + 1, 1)

        @pl.when(b0 + 3 < NBK)
        def _():
            idx_fire(b0 + 3, 1)

        @pl.when(b0 + 2 < NBK)
        def _():
            gat_fire(b0 + 2, 0)

        return carry

    lax.fori_loop(0, (NBK + 1) // 2, body, 0)
    plsc.subcore_barrier()
    pltpu.sync_copy(
        agg_sp.at[pl.ds(s * ROWS_PER_TILE, ROWS_PER_TILE), :],
        out_hbm.at[c, pl.ds(s * ROWS_PER_TILE, ROWS_PER_TILE), :],
    )


def _enc_body(x_ref, cat_ref, wn_ref, bn_ref, win_n_ref, win_c_ref, bin_ref,
              out_ref):
    cat = cat_ref[...].reshape(BLK, NUM_CAT * EMBED)
    num = jnp.maximum(
        jnp.dot(x_ref[...], wn_ref[...], preferred_element_type=jnp.float32)
        + bn_ref[...], 0.0)
    h = (jnp.dot(num, win_n_ref[...], preferred_element_type=jnp.float32)
         + jnp.dot(cat, win_c_ref[...],
                   preferred_element_type=jnp.float32)
         + bin_ref[...])
    h = jnp.maximum(h, 0.0)
    out_ref[0] = h[:, :EMBED]
    out_ref[1] = h[:, EMBED:]


_encoder = pl.pallas_call(
    _enc_body,
    grid=(NBLK,),
    in_specs=[
        pl.BlockSpec((BLK, NUM_NUMERIC), lambda i: (i, 0)),
        pl.BlockSpec((2 * BLK, 128), lambda i: (i, 0)),
        pl.BlockSpec((NUM_NUMERIC, EMBED), lambda i: (0, 0)),
        pl.BlockSpec((1, EMBED), lambda i: (0, 0)),
        pl.BlockSpec((EMBED, HIDDEN), lambda i: (0, 0)),
        pl.BlockSpec((NUM_CAT * EMBED, HIDDEN), lambda i: (0, 0)),
        pl.BlockSpec((1, HIDDEN), lambda i: (0, 0)),
    ],
    out_specs=pl.BlockSpec((2, BLK, EMBED), lambda i: (0, i, 0)),
    out_shape=jax.ShapeDtypeStruct((2, NP, EMBED), jnp.float32),
)


def _layer_body(with_head, agg_ref, deg_ref, hres_ref, wg_ref, bg_ref,
                gam_ref, bet_ref, *rest):
    if with_head:
        wo_ref, bo_ref, out_ref, outc_ref = rest
    else:
        (out_ref,) = rest
    agg = jnp.concatenate([agg_ref[0], agg_ref[1]], axis=1)
    deg = deg_ref[0][:, :1] + deg_ref[1][:, :1]
    hmp = agg * (1.0 / jnp.maximum(deg, 1.0))
    hr = jnp.concatenate([hres_ref[0], hres_ref[1]], axis=1)
    t = jnp.maximum(
        jnp.dot(hmp, wg_ref[...], preferred_element_type=jnp.float32)
        + bg_ref[...], 0.0) + hr
    mu = jnp.mean(t, axis=1, keepdims=True)
    var = jnp.mean((t - mu) ** 2, axis=1, keepdims=True)
    h = (t - mu) * lax.rsqrt(var + 1e-5) * gam_ref[...] + bet_ref[...]
    out_ref[0] = h[:, :EMBED]
    out_ref[1] = h[:, EMBED:]
    if with_head:
        outc_ref[...] = (
            jnp.dot(h, wo_ref[...], preferred_element_type=jnp.float32)
            + bo_ref[...])


def _make_layer(with_head):
    in_specs = [
        pl.BlockSpec((2, BLK, EMBED), lambda i: (0, i, 0)),
        pl.BlockSpec((2, BLK, DEGW), lambda i: (0, i, 0)),
        pl.BlockSpec((2, BLK, EMBED), lambda i: (0, i, 0)),
        pl.BlockSpec((HIDDEN, HIDDEN), lambda i: (0, 0)),
        pl.BlockSpec((1, HIDDEN), lambda i: (0, 0)),
        pl.BlockSpec((1, HIDDEN), lambda i: (0, 0)),
        pl.BlockSpec((1, HIDDEN), lambda i: (0, 0)),
    ]
    out_specs = pl.BlockSpec((2, BLK, EMBED), lambda i: (0, i, 0))
    out_shape = jax.ShapeDtypeStruct((2, NP, EMBED), jnp.float32)
    if with_head:
        in_specs += [
            pl.BlockSpec((HIDDEN, 1), lambda i: (0, 0)),
            pl.BlockSpec((1, 1), lambda i: (0, 0)),
        ]
        out_specs = [out_specs, pl.BlockSpec((BLK, 1), lambda i: (i, 0))]
        out_shape = [out_shape, jax.ShapeDtypeStruct((NP, 1), jnp.float32)]
    return pl.pallas_call(
        functools.partial(_layer_body, with_head),
        grid=(NBLK,),
        in_specs=in_specs,
        out_specs=out_specs,
        out_shape=out_shape,
    )


_layer1 = _make_layer(False)
_layer2 = _make_layer(True)


def kernel(x_num, x_cat, edge_index, W_num, b_num, emb_tables, W_in, b_in,
           W_g1, b_g1, W_g2, b_g2, gamma1, beta1, gamma2, beta2, W_out, b_out):
    f32 = jnp.float32
    ids = jnp.clip(x_cat, 0, CAT_SIZE) + (
        jnp.arange(NUM_CAT, dtype=jnp.int32) * (CAT_SIZE + 1))
    ids = jnp.pad(ids, ((0, NP - N), (0, 0))).reshape(-1)
    flat_tab = emb_tables.reshape(NUM_CAT * (CAT_SIZE + 1), EMBED)
    x_num_p = jnp.pad(x_num, ((0, NP - N), (0, 0)))

    src = jnp.pad(edge_index[0], (0, EP - E))
    dst = jnp.pad(edge_index[1], (0, EP - E), constant_values=N)
    src2 = jnp.stack([src, src + NP])

    zeros_rows = jnp.zeros((ROWS_PER_TILE, EMBED), f32)
    zeros_col = jnp.zeros((ROWS_PER_TILE, DEGW), f32)
    ones_col = jnp.ones((C, DEGW), f32)

    cat_embed = _emb_gather(flat_tab, ids).reshape(2 * NP, 128)
    h0 = _encoder(x_num_p, cat_embed, W_num, b_num.reshape(1, EMBED),
                  W_in[:EMBED], W_in[EMBED:], b_in.reshape(1, HIDDEN))
    deg2 = _deg_count(dst, ones_col, zeros_col)

    agg1 = _edge_agg(h0.reshape(2 * NP, EMBED), src2, dst, zeros_rows)
    h1 = _layer1(agg1, deg2, h0, W_g1, b_g1.reshape(1, HIDDEN),
                 gamma1.reshape(1, HIDDEN), beta1.reshape(1, HIDDEN))

    agg2 = _edge_agg(h1.reshape(2 * NP, EMBED), src2, dst, zeros_rows)
    _, outc = _layer2(agg2, deg2, h1, W_g2, b_g2.reshape(1, HIDDEN),
                      gamma2.reshape(1, HIDDEN), beta2.reshape(1, HIDDEN),
                      W_out, b_out.reshape(1, 1))
    return outc[:N, 0]
